# baseline (speedup 1.0000x reference)
import functools

import jax
import jax.numpy as jnp
from jax.experimental import pallas as pl
from jax.experimental.pallas import tpu as pltpu

B, D, H = 1024, 128, 65536
BM = 128
HBLK = 8192
NB = B // BM
NH = H // HBLK
CCHUNK = 4096
NCHUNK = H // CCHUNK
GBLK = HBLK // 64
RW = NH * GBLK
RBLKS = RW // 512


def _body(k_ref, x_ref, wenc_ref, benc_ref, wdec_ref, o_ref, recon_ref,
          feat, rmax, tbits):
    p = pl.program_id(1)
    h = pl.program_id(2)

    @pl.when(p == 0)
    def _encode():
        acc = jax.lax.dot_general(
            x_ref[...], wenc_ref[...], (((1,), (0,)), ((), ())),
            preferred_element_type=jnp.float32)
        fblk = jnp.maximum(acc + benc_ref[...], 0.0)
        feat[:, pl.ds(h * HBLK, HBLK)] = fblk
        r = fblk
        for _ in range(6):
            half = r.shape[1] // 2
            r = jnp.maximum(r[:, :half], r[:, half:])
        rmax[:, pl.ds(h * GBLK, GBLK)] = r

    @pl.when((p == 0) & (h == NH - 1))
    def _select():
        kk = k_ref[0].astype(jnp.float32)

        def count_ge(cand_f):
            def chunk(i, acc):
                fb = feat[:, pl.ds(i * CCHUNK, CCHUNK)]
                ge = jnp.where(fb >= cand_f, 1.0, 0.0)
                return acc + jnp.sum(ge, axis=1, keepdims=True)
            return jax.lax.fori_loop(0, NCHUNK, chunk,
                                     jnp.zeros((BM, 1), jnp.float32))

        def count_rmax(cand_f):
            def chunk(i, acc):
                rb = rmax[:, pl.ds(i * 512, 512)]
                ge = jnp.where(rb >= cand_f, 1.0, 0.0)
                return acc + jnp.sum(ge, axis=1, keepdims=True)
            return jax.lax.fori_loop(0, RBLKS, chunk,
                                     jnp.zeros((BM, 1), jnp.float32))

        def bits_f(b):
            return jax.lax.bitcast_convert_type(b, jnp.float32)

        def hybrid(count_fn, lob, cnt_lo, hib, cnt_hi):
            def done_of(st):
                _, lob, cnt_lo, hib, _ = st
                return (cnt_lo == kk) | (hib - lob <= 1)

            def cond(st):
                return ~jnp.all(done_of(st))

            def body(st):
                it, lob, cnt_lo, hib, cnt_hi = st
                done = done_of(st)
                lo_f, hi_f = bits_f(lob), bits_f(hib)
                frac = (cnt_lo - kk) / jnp.maximum(cnt_lo - cnt_hi, 1.0)
                interp = jax.lax.bitcast_convert_type(
                    lo_f + (hi_f - lo_f) * frac, jnp.int32)
                bisect = lob + jnp.maximum((hib - lob) >> 1, 1)
                cand = jnp.where(it % 2 == 0, interp, bisect)
                cand = jnp.clip(cand, lob + 1,
                                jnp.maximum(hib - 1, lob + 1))
                cand = jnp.where(done, lob, cand)
                cnt = count_fn(bits_f(cand))
                ge = cnt >= kk
                keep = done
                return (it + 1,
                        jnp.where(keep | ~ge, lob, cand),
                        jnp.where(keep | ~ge, cnt_lo, cnt),
                        jnp.where(keep | ge, hib, cand),
                        jnp.where(keep | ge, cnt_hi, cnt))

            st = (jnp.int32(0), lob, cnt_lo, hib, cnt_hi)
            st = jax.lax.while_loop(cond, body, st)
            return st[1]

        def rowmax_bits():
            def chunk(i, acc):
                rb = rmax[:, pl.ds(i * 512, 512)]
                return jnp.maximum(acc, jnp.max(rb, axis=1, keepdims=True))
            m = jax.lax.fori_loop(0, RBLKS, chunk,
                                  jnp.zeros((BM, 1), jnp.float32))
            return jax.lax.bitcast_convert_type(m, jnp.int32)

        zi = jnp.zeros((BM, 1), jnp.int32)
        zf = jnp.zeros((BM, 1), jnp.float32)
        hib0 = rowmax_bits() + 1

        u_lo = hybrid(count_rmax, zi, jnp.full((BM, 1), float(RW)), hib0, zf)

        cnt_lo0 = count_ge(bits_f(u_lo))
        tbits[...] = hybrid(count_ge, u_lo, cnt_lo0, hib0, zf)

    @pl.when(p == 1)
    def _mask_decode():
        t = jax.lax.bitcast_convert_type(tbits[...], jnp.float32)
        fb = feat[:, pl.ds(h * HBLK, HBLK)]
        masked = jnp.where(fb >= t, fb, 0.0)
        o_ref[...] = masked
        partial = jax.lax.dot_general(
            masked, wdec_ref[...], (((1,), (0,)), ((), ())),
            preferred_element_type=jnp.float32)

        @pl.when(h == 0)
        def _init():
            recon_ref[...] = partial

        @pl.when(h != 0)
        def _accum():
            recon_ref[...] = recon_ref[...] + partial


def _im_x(b, p, h, k_ref):
    return (b, 0)


def _im_wenc(b, p, h, k_ref):
    return (0, jnp.where(p == 0, h, NH - 1))


def _im_benc(b, p, h, k_ref):
    return (0, jnp.where(p == 0, h, NH - 1))


def _im_wdec(b, p, h, k_ref):
    return (jnp.where(p == 1, h, 0), 0)


def _im_out(b, p, h, k_ref):
    return (b, jnp.where(p == 1, h, 0))


def _im_recon(b, p, h, k_ref):
    return (b, 0)


@jax.jit
def kernel(x, W_enc, b_enc, W_dec, k):
    k_arr = jnp.asarray(k, jnp.int32).reshape((1,))
    b_enc2d = b_enc.reshape((1, H))

    grid_spec = pltpu.PrefetchScalarGridSpec(
        num_scalar_prefetch=1,
        grid=(NB, 2, NH),
        in_specs=[
            pl.BlockSpec((BM, D), _im_x),
            pl.BlockSpec((D, HBLK), _im_wenc),
            pl.BlockSpec((1, HBLK), _im_benc),
            pl.BlockSpec((HBLK, D), _im_wdec),
        ],
        out_specs=[
            pl.BlockSpec((BM, HBLK), _im_out),
            pl.BlockSpec((BM, D), _im_recon),
        ],
        scratch_shapes=[
            pltpu.VMEM((BM, H), jnp.float32),
            pltpu.VMEM((BM, RW), jnp.float32),
            pltpu.VMEM((BM, 1), jnp.int32),
        ],
    )

    out = pl.pallas_call(
        _body,
        grid_spec=grid_spec,
        out_shape=[
            jax.ShapeDtypeStruct((B, H), jnp.float32),
            jax.ShapeDtypeStruct((B, D), jnp.float32),
        ],
        compiler_params=pltpu.CompilerParams(
            dimension_semantics=("arbitrary", "arbitrary", "arbitrary"),
        ),
    )(k_arr, x, W_enc, b_enc2d, W_dec)
    return (out[0], out[1])


# speedup vs baseline: 1.0675x; 1.0675x over previous
import functools

import jax
import jax.numpy as jnp
from jax.experimental import pallas as pl
from jax.experimental.pallas import tpu as pltpu

B, D, H = 1024, 128, 65536
BM = 128
HBLK = 4096
NB = B // BM
NH = H // HBLK
CCHUNK = 8192
NCHUNK = H // CCHUNK
GBLK = HBLK // 16
RW = NH * GBLK
RBLKS = RW // 2048


def _body(k_ref, x_ref, wenc_ref, benc_ref, wdec_ref, o_ref, recon_ref,
          feat, rmax, tbits):
    p = pl.program_id(1)
    h = pl.program_id(2)

    @pl.when(p == 0)
    def _encode():
        acc = jax.lax.dot_general(
            x_ref[...], wenc_ref[...], (((1,), (0,)), ((), ())),
            preferred_element_type=jnp.float32)
        fblk = jnp.maximum(acc + benc_ref[...], 0.0)
        feat[:, pl.ds(h * HBLK, HBLK)] = fblk
        r = fblk
        for _ in range(4):
            half = r.shape[1] // 2
            r = jnp.maximum(r[:, :half], r[:, half:])
        rmax[:, pl.ds(h * GBLK, GBLK)] = r

    @pl.when((p == 0) & (h == NH - 1))
    def _select():
        kk = k_ref[0].astype(jnp.float32)

        def count_ge(cand_f):
            def chunk(i, acc):
                fb = feat[:, pl.ds(i * CCHUNK, CCHUNK)]
                ge = jnp.where(fb >= cand_f, 1.0, 0.0)
                return acc + jnp.sum(ge, axis=1, keepdims=True)
            return jax.lax.fori_loop(0, NCHUNK, chunk,
                                     jnp.zeros((BM, 1), jnp.float32))

        def count_rmax(cand_f):
            def chunk(i, acc):
                rb = rmax[:, pl.ds(i * 2048, 2048)]
                ge = jnp.where(rb >= cand_f, 1.0, 0.0)
                return acc + jnp.sum(ge, axis=1, keepdims=True)
            return jax.lax.fori_loop(0, RBLKS, chunk,
                                     jnp.zeros((BM, 1), jnp.float32))

        def bits_f(b):
            return jax.lax.bitcast_convert_type(b, jnp.float32)

        def hybrid(count_fn, lob, cnt_lo, hib, cnt_hi):
            def done_of(st):
                _, lob, cnt_lo, hib, _ = st
                return (cnt_lo == kk) | (hib - lob <= 1)

            def cond(st):
                return ~jnp.all(done_of(st))

            def body(st):
                it, lob, cnt_lo, hib, cnt_hi = st
                done = done_of(st)
                lo_f, hi_f = bits_f(lob), bits_f(hib)
                frac = (cnt_lo - kk) / jnp.maximum(cnt_lo - cnt_hi, 1.0)
                interp = jax.lax.bitcast_convert_type(
                    lo_f + (hi_f - lo_f) * frac, jnp.int32)
                bisect = lob + jnp.maximum((hib - lob) >> 1, 1)
                cand = jnp.where(it % 2 == 0, interp, bisect)
                cand = jnp.clip(cand, lob + 1,
                                jnp.maximum(hib - 1, lob + 1))
                cand = jnp.where(done, lob, cand)
                cnt = count_fn(bits_f(cand))
                ge = cnt >= kk
                keep = done
                return (it + 1,
                        jnp.where(keep | ~ge, lob, cand),
                        jnp.where(keep | ~ge, cnt_lo, cnt),
                        jnp.where(keep | ge, hib, cand),
                        jnp.where(keep | ge, cnt_hi, cnt))

            st = (jnp.int32(0), lob, cnt_lo, hib, cnt_hi)
            st = jax.lax.while_loop(cond, body, st)
            return st[1]

        def rowmax_bits():
            def chunk(i, acc):
                rb = rmax[:, pl.ds(i * 2048, 2048)]
                return jnp.maximum(acc, jnp.max(rb, axis=1, keepdims=True))
            m = jax.lax.fori_loop(0, RBLKS, chunk,
                                  jnp.zeros((BM, 1), jnp.float32))
            return jax.lax.bitcast_convert_type(m, jnp.int32)

        zi = jnp.zeros((BM, 1), jnp.int32)
        zf = jnp.zeros((BM, 1), jnp.float32)
        hib0 = rowmax_bits() + 1

        u_lo = hybrid(count_rmax, zi, jnp.full((BM, 1), float(RW)), hib0, zf)

        cnt_lo0 = count_ge(bits_f(u_lo))
        tbits[...] = hybrid(count_ge, u_lo, cnt_lo0, hib0, zf)

    @pl.when(p == 1)
    def _mask_decode():
        t = jax.lax.bitcast_convert_type(tbits[...], jnp.float32)
        fb = feat[:, pl.ds(h * HBLK, HBLK)]
        masked = jnp.where(fb >= t, fb, 0.0)
        o_ref[...] = masked
        partial = jax.lax.dot_general(
            masked, wdec_ref[...], (((1,), (0,)), ((), ())),
            preferred_element_type=jnp.float32)

        @pl.when(h == 0)
        def _init():
            recon_ref[...] = partial

        @pl.when(h != 0)
        def _accum():
            recon_ref[...] = recon_ref[...] + partial


def _im_x(b, p, h, k_ref):
    return (b, 0)


def _im_wenc(b, p, h, k_ref):
    return (0, jnp.where(p == 0, h, NH - 1))


def _im_benc(b, p, h, k_ref):
    return (0, jnp.where(p == 0, h, NH - 1))


def _im_wdec(b, p, h, k_ref):
    return (jnp.where(p == 1, h, 0), 0)


def _im_out(b, p, h, k_ref):
    return (b, jnp.where(p == 1, h, 0))


def _im_recon(b, p, h, k_ref):
    return (b, 0)


@jax.jit
def kernel(x, W_enc, b_enc, W_dec, k):
    k_arr = jnp.asarray(k, jnp.int32).reshape((1,))
    b_enc2d = b_enc.reshape((1, H))

    grid_spec = pltpu.PrefetchScalarGridSpec(
        num_scalar_prefetch=1,
        grid=(NB, 2, NH),
        in_specs=[
            pl.BlockSpec((BM, D), _im_x),
            pl.BlockSpec((D, HBLK), _im_wenc),
            pl.BlockSpec((1, HBLK), _im_benc),
            pl.BlockSpec((HBLK, D), _im_wdec),
        ],
        out_specs=[
            pl.BlockSpec((BM, HBLK), _im_out),
            pl.BlockSpec((BM, D), _im_recon),
        ],
        scratch_shapes=[
            pltpu.VMEM((BM, H), jnp.float32),
            pltpu.VMEM((BM, RW), jnp.float32),
            pltpu.VMEM((BM, 1), jnp.int32),
        ],
    )

    out = pl.pallas_call(
        _body,
        grid_spec=grid_spec,
        out_shape=[
            jax.ShapeDtypeStruct((B, H), jnp.float32),
            jax.ShapeDtypeStruct((B, D), jnp.float32),
        ],
        compiler_params=pltpu.CompilerParams(
            dimension_semantics=("arbitrary", "arbitrary", "arbitrary"),
        ),
    )(k_arr, x, W_enc, b_enc2d, W_dec)
    return (out[0], out[1])


# speedup vs baseline: 1.0947x; 1.0255x over previous
import functools

import jax
import jax.numpy as jnp
from jax.experimental import pallas as pl
from jax.experimental.pallas import tpu as pltpu

B, D, H = 1024, 128, 65536
BM = 128
HBLK = 4096
NB = B // BM
NH = H // HBLK
CCHUNK = 16384
NCHUNK = H // CCHUNK
GBLK = HBLK // 16
RW = NH * GBLK
RBLKS = RW // 2048


def _body(k_ref, x_ref, wenc_ref, benc_ref, wdec_ref, o_ref, recon_ref,
          feat, rmax, tbits):
    p = pl.program_id(1)
    h = pl.program_id(2)

    @pl.when(p == 0)
    def _encode():
        acc = jax.lax.dot_general(
            x_ref[...], wenc_ref[...], (((1,), (0,)), ((), ())),
            preferred_element_type=jnp.float32)
        fblk = jnp.maximum(acc + benc_ref[...], 0.0)
        feat[:, pl.ds(h * HBLK, HBLK)] = fblk
        r = fblk
        for _ in range(4):
            half = r.shape[1] // 2
            r = jnp.maximum(r[:, :half], r[:, half:])
        rmax[:, pl.ds(h * GBLK, GBLK)] = r

    @pl.when((p == 0) & (h == NH - 1))
    def _select():
        kk = k_ref[0].astype(jnp.float32)

        def count_ge(cand_f):
            def chunk(i, acc):
                fb = feat[:, pl.ds(i * CCHUNK, CCHUNK)]
                ge = jnp.where(fb >= cand_f, 1.0, 0.0)
                return acc + jnp.sum(ge, axis=1, keepdims=True)
            return jax.lax.fori_loop(0, NCHUNK, chunk,
                                     jnp.zeros((BM, 1), jnp.float32))

        def count_rmax(cand_f):
            def chunk(i, acc):
                rb = rmax[:, pl.ds(i * 2048, 2048)]
                ge = jnp.where(rb >= cand_f, 1.0, 0.0)
                return acc + jnp.sum(ge, axis=1, keepdims=True)
            return jax.lax.fori_loop(0, RBLKS, chunk,
                                     jnp.zeros((BM, 1), jnp.float32))

        def bits_f(b):
            return jax.lax.bitcast_convert_type(b, jnp.float32)

        def hybrid(count_fn, lob, cnt_lo, hib, cnt_hi):
            def done_of(st):
                _, lob, cnt_lo, hib, _ = st
                return (cnt_lo == kk) | (hib - lob <= 1)

            def cond(st):
                return ~jnp.all(done_of(st))

            def body(st):
                it, lob, cnt_lo, hib, cnt_hi = st
                done = done_of(st)
                lo_f, hi_f = bits_f(lob), bits_f(hib)
                frac = (cnt_lo - kk) / jnp.maximum(cnt_lo - cnt_hi, 1.0)
                interp = jax.lax.bitcast_convert_type(
                    lo_f + (hi_f - lo_f) * frac, jnp.int32)
                bisect = lob + jnp.maximum((hib - lob) >> 1, 1)
                cand = jnp.where(it % 2 == 0, interp, bisect)
                cand = jnp.clip(cand, lob + 1,
                                jnp.maximum(hib - 1, lob + 1))
                cand = jnp.where(done, lob, cand)
                cnt = count_fn(bits_f(cand))
                ge = cnt >= kk
                keep = done
                return (it + 1,
                        jnp.where(keep | ~ge, lob, cand),
                        jnp.where(keep | ~ge, cnt_lo, cnt),
                        jnp.where(keep | ge, hib, cand),
                        jnp.where(keep | ge, cnt_hi, cnt))

            st = (jnp.int32(0), lob, cnt_lo, hib, cnt_hi)
            st = jax.lax.while_loop(cond, body, st)
            return st[1]

        def rowmax_bits():
            def chunk(i, acc):
                rb = rmax[:, pl.ds(i * 2048, 2048)]
                return jnp.maximum(acc, jnp.max(rb, axis=1, keepdims=True))
            m = jax.lax.fori_loop(0, RBLKS, chunk,
                                  jnp.zeros((BM, 1), jnp.float32))
            return jax.lax.bitcast_convert_type(m, jnp.int32)

        zi = jnp.zeros((BM, 1), jnp.int32)
        zf = jnp.zeros((BM, 1), jnp.float32)
        hib0 = rowmax_bits() + 1

        u_lo = hybrid(count_rmax, zi, jnp.full((BM, 1), float(RW)), hib0, zf)

        cnt_lo0 = count_ge(bits_f(u_lo))
        tbits[...] = hybrid(count_ge, u_lo, cnt_lo0, hib0, zf)

    @pl.when(p == 1)
    def _mask_decode():
        t = jax.lax.bitcast_convert_type(tbits[...], jnp.float32)
        fb = feat[:, pl.ds(h * HBLK, HBLK)]
        masked = jnp.where(fb >= t, fb, 0.0)
        o_ref[...] = masked
        partial = jax.lax.dot_general(
            masked, wdec_ref[...], (((1,), (0,)), ((), ())),
            preferred_element_type=jnp.float32)

        @pl.when(h == 0)
        def _init():
            recon_ref[...] = partial

        @pl.when(h != 0)
        def _accum():
            recon_ref[...] = recon_ref[...] + partial


def _im_x(b, p, h, k_ref):
    return (b, 0)


def _im_wenc(b, p, h, k_ref):
    return (0, jnp.where(p == 0, h, NH - 1))


def _im_benc(b, p, h, k_ref):
    return (0, jnp.where(p == 0, h, NH - 1))


def _im_wdec(b, p, h, k_ref):
    return (jnp.where(p == 1, h, 0), 0)


def _im_out(b, p, h, k_ref):
    return (b, jnp.where(p == 1, h, 0))


def _im_recon(b, p, h, k_ref):
    return (b, 0)


@jax.jit
def kernel(x, W_enc, b_enc, W_dec, k):
    k_arr = jnp.asarray(k, jnp.int32).reshape((1,))
    b_enc2d = b_enc.reshape((1, H))

    grid_spec = pltpu.PrefetchScalarGridSpec(
        num_scalar_prefetch=1,
        grid=(NB, 2, NH),
        in_specs=[
            pl.BlockSpec((BM, D), _im_x),
            pl.BlockSpec((D, HBLK), _im_wenc),
            pl.BlockSpec((1, HBLK), _im_benc),
            pl.BlockSpec((HBLK, D), _im_wdec),
        ],
        out_specs=[
            pl.BlockSpec((BM, HBLK), _im_out),
            pl.BlockSpec((BM, D), _im_recon),
        ],
        scratch_shapes=[
            pltpu.VMEM((BM, H), jnp.float32),
            pltpu.VMEM((BM, RW), jnp.float32),
            pltpu.VMEM((BM, 1), jnp.int32),
        ],
    )

    out = pl.pallas_call(
        _body,
        grid_spec=grid_spec,
        out_shape=[
            jax.ShapeDtypeStruct((B, H), jnp.float32),
            jax.ShapeDtypeStruct((B, D), jnp.float32),
        ],
        compiler_params=pltpu.CompilerParams(
            dimension_semantics=("arbitrary", "arbitrary", "arbitrary"),
        ),
    )(k_arr, x, W_enc, b_enc2d, W_dec)
    return (out[0], out[1])


# speedup vs baseline: 1.1037x; 1.0082x over previous
import functools

import jax
import jax.numpy as jnp
from jax.experimental import pallas as pl
from jax.experimental.pallas import tpu as pltpu

B, D, H = 1024, 128, 65536
BM = 128
HBLK = 4096
NB = B // BM
NH = H // HBLK
CCHUNK = 32768
NCHUNK = H // CCHUNK
GBLK = HBLK // 16
RW = NH * GBLK
RBLKS = RW // 2048


def _body(k_ref, x_ref, wenc_ref, benc_ref, wdec_ref, o_ref, recon_ref,
          feat, rmax, tbits):
    p = pl.program_id(1)
    h = pl.program_id(2)

    @pl.when(p == 0)
    def _encode():
        acc = jax.lax.dot_general(
            x_ref[...], wenc_ref[...], (((1,), (0,)), ((), ())),
            preferred_element_type=jnp.float32)
        fblk = jnp.maximum(acc + benc_ref[...], 0.0)
        feat[:, pl.ds(h * HBLK, HBLK)] = fblk
        r = fblk
        for _ in range(4):
            half = r.shape[1] // 2
            r = jnp.maximum(r[:, :half], r[:, half:])
        rmax[:, pl.ds(h * GBLK, GBLK)] = r

    @pl.when((p == 0) & (h == NH - 1))
    def _select():
        kk = k_ref[0].astype(jnp.float32)

        def count_ge(cand_f):
            def chunk(i, acc):
                fb = feat[:, pl.ds(i * CCHUNK, CCHUNK)]
                ge = jnp.where(fb >= cand_f, 1.0, 0.0)
                return acc + jnp.sum(ge, axis=1, keepdims=True)
            return jax.lax.fori_loop(0, NCHUNK, chunk,
                                     jnp.zeros((BM, 1), jnp.float32))

        def count_rmax(cand_f):
            def chunk(i, acc):
                rb = rmax[:, pl.ds(i * 2048, 2048)]
                ge = jnp.where(rb >= cand_f, 1.0, 0.0)
                return acc + jnp.sum(ge, axis=1, keepdims=True)
            return jax.lax.fori_loop(0, RBLKS, chunk,
                                     jnp.zeros((BM, 1), jnp.float32))

        def bits_f(b):
            return jax.lax.bitcast_convert_type(b, jnp.float32)

        def hybrid(count_fn, lob, cnt_lo, hib, cnt_hi):
            def done_of(st):
                _, lob, cnt_lo, hib, _ = st
                return (cnt_lo == kk) | (hib - lob <= 1)

            def cond(st):
                return ~jnp.all(done_of(st))

            def body(st):
                it, lob, cnt_lo, hib, cnt_hi = st
                done = done_of(st)
                lo_f, hi_f = bits_f(lob), bits_f(hib)
                frac = (cnt_lo - kk) / jnp.maximum(cnt_lo - cnt_hi, 1.0)
                interp = jax.lax.bitcast_convert_type(
                    lo_f + (hi_f - lo_f) * frac, jnp.int32)
                bisect = lob + jnp.maximum((hib - lob) >> 1, 1)
                cand = jnp.where(it % 2 == 0, interp, bisect)
                cand = jnp.clip(cand, lob + 1,
                                jnp.maximum(hib - 1, lob + 1))
                cand = jnp.where(done, lob, cand)
                cnt = count_fn(bits_f(cand))
                ge = cnt >= kk
                keep = done
                return (it + 1,
                        jnp.where(keep | ~ge, lob, cand),
                        jnp.where(keep | ~ge, cnt_lo, cnt),
                        jnp.where(keep | ge, hib, cand),
                        jnp.where(keep | ge, cnt_hi, cnt))

            st = (jnp.int32(0), lob, cnt_lo, hib, cnt_hi)
            st = jax.lax.while_loop(cond, body, st)
            return st[1]

        def rowmax_bits():
            def chunk(i, acc):
                rb = rmax[:, pl.ds(i * 2048, 2048)]
                return jnp.maximum(acc, jnp.max(rb, axis=1, keepdims=True))
            m = jax.lax.fori_loop(0, RBLKS, chunk,
                                  jnp.zeros((BM, 1), jnp.float32))
            return jax.lax.bitcast_convert_type(m, jnp.int32)

        zi = jnp.zeros((BM, 1), jnp.int32)
        zf = jnp.zeros((BM, 1), jnp.float32)
        hib0 = rowmax_bits() + 1

        u_lo = hybrid(count_rmax, zi, jnp.full((BM, 1), float(RW)), hib0, zf)

        cnt_lo0 = count_ge(bits_f(u_lo))
        tbits[...] = hybrid(count_ge, u_lo, cnt_lo0, hib0, zf)

    @pl.when(p == 1)
    def _mask_decode():
        t = jax.lax.bitcast_convert_type(tbits[...], jnp.float32)
        fb = feat[:, pl.ds(h * HBLK, HBLK)]
        masked = jnp.where(fb >= t, fb, 0.0)
        o_ref[...] = masked
        partial = jax.lax.dot_general(
            masked, wdec_ref[...], (((1,), (0,)), ((), ())),
            preferred_element_type=jnp.float32)

        @pl.when(h == 0)
        def _init():
            recon_ref[...] = partial

        @pl.when(h != 0)
        def _accum():
            recon_ref[...] = recon_ref[...] + partial


def _im_x(b, p, h, k_ref):
    return (b, 0)


def _im_wenc(b, p, h, k_ref):
    return (0, jnp.where(p == 0, h, NH - 1))


def _im_benc(b, p, h, k_ref):
    return (0, jnp.where(p == 0, h, NH - 1))


def _im_wdec(b, p, h, k_ref):
    return (jnp.where(p == 1, h, 0), 0)


def _im_out(b, p, h, k_ref):
    return (b, jnp.where(p == 1, h, 0))


def _im_recon(b, p, h, k_ref):
    return (b, 0)


@jax.jit
def kernel(x, W_enc, b_enc, W_dec, k):
    k_arr = jnp.asarray(k, jnp.int32).reshape((1,))
    b_enc2d = b_enc.reshape((1, H))

    grid_spec = pltpu.PrefetchScalarGridSpec(
        num_scalar_prefetch=1,
        grid=(NB, 2, NH),
        in_specs=[
            pl.BlockSpec((BM, D), _im_x),
            pl.BlockSpec((D, HBLK), _im_wenc),
            pl.BlockSpec((1, HBLK), _im_benc),
            pl.BlockSpec((HBLK, D), _im_wdec),
        ],
        out_specs=[
            pl.BlockSpec((BM, HBLK), _im_out),
            pl.BlockSpec((BM, D), _im_recon),
        ],
        scratch_shapes=[
            pltpu.VMEM((BM, H), jnp.float32),
            pltpu.VMEM((BM, RW), jnp.float32),
            pltpu.VMEM((BM, 1), jnp.int32),
        ],
    )

    out = pl.pallas_call(
        _body,
        grid_spec=grid_spec,
        out_shape=[
            jax.ShapeDtypeStruct((B, H), jnp.float32),
            jax.ShapeDtypeStruct((B, D), jnp.float32),
        ],
        compiler_params=pltpu.CompilerParams(
            dimension_semantics=("arbitrary", "arbitrary", "arbitrary"),
        ),
    )(k_arr, x, W_enc, b_enc2d, W_dec)
    return (out[0], out[1])
